# baseline (speedup 1.0000x reference)
import functools

import jax
import jax.numpy as jnp
from jax import lax
from jax.experimental import pallas as pl
from jax.experimental.pallas import tpu as pltpu
from jax.experimental.pallas import tpu_sc as plsc


def _build_sc_scatter(batch_size, n_queries, n_keys, lanes):
    mesh = plsc.VectorSubcoreMesh(core_axis_name="c", subcore_axis_name="s")

    @functools.partial(
        pl.kernel,
        mesh=mesh,
        out_type=(),
        scratch_types=[
            pltpu.VMEM((lanes,), jnp.int32),
            pltpu.VMEM((lanes, lanes), jnp.float32),
            pltpu.SemaphoreType.DMA,
            pltpu.SemaphoreType.DMA,
        ],
    )
    def sc_kernel(idx_hbm, out_hbm, idx_v, chunk_v, isem, csem):
        wid = lax.axis_index("s") * 2 + lax.axis_index("c")
        qq = wid // 2
        b0 = (wid % 2) * lanes
        idx_copy = pltpu.async_copy(idx_hbm.at[qq, pl.ds(b0, lanes)], idx_v, isem)

        lane_iota = lax.iota(jnp.int32, lanes)
        idx_copy.wait()
        iv = jnp.maximum(idx_v[...], 0)
        for j in range(lanes):
            m = iv[j]
            chunk_v[j, pl.ds(0, lanes)] = (
                1 - jnp.minimum(jnp.abs(lane_iota - m % lanes), 1)
            ).astype(jnp.float32)

        copies = [
            pltpu.async_copy(
                chunk_v.at[j],
                out_hbm.at[b0 + j, qq, pl.ds((iv[j] // lanes) * lanes, lanes)],
                csem,
            )
            for j in range(lanes)
        ]
        for cp in copies:
            cp.wait()

    return sc_kernel


def _tc_zero(batch_size, n_queries, n_keys):
    def body(out_ref):
        out_ref[...] = jnp.zeros_like(out_ref)

    bb = 4
    return pl.pallas_call(
        body,
        out_shape=jax.ShapeDtypeStruct((batch_size, n_queries, n_keys), jnp.float32),
        grid=(batch_size // bb,),
        out_specs=pl.BlockSpec((bb, n_queries, n_keys), lambda b: (b, 0, 0)),
    )()


def kernel(keys, queries, step, provided_attention):
    batch_size, n_queries, _ = queries.shape
    n_keys = keys.shape[1]
    lanes = 16

    idx_t = provided_attention.T.astype(jnp.int32)
    acc = jax.new_ref(_tc_zero(batch_size, n_queries, n_keys))
    _build_sc_scatter(batch_size, n_queries, n_keys, lanes)(idx_t, acc)
    return acc[...]


# speedup vs baseline: 1.0076x; 1.0076x over previous
import functools

import jax
import jax.numpy as jnp
from jax import lax
from jax.experimental import pallas as pl
from jax.experimental.pallas import tpu as pltpu
from jax.experimental.pallas import tpu_sc as plsc


def _build_sc_scatter(batch_size, n_queries, n_keys, lanes):
    mesh = plsc.VectorSubcoreMesh(core_axis_name="c", subcore_axis_name="s")

    @functools.partial(
        pl.kernel,
        mesh=mesh,
        out_type=(),
        scratch_types=[
            pltpu.VMEM((lanes,), jnp.int32),
            pltpu.VMEM((lanes, lanes), jnp.float32),
            pltpu.SemaphoreType.DMA,
            pltpu.SemaphoreType.DMA,
        ],
    )
    def sc_kernel(idx_hbm, out_hbm, idx_v, chunk_v, isem, csem):
        wid = lax.axis_index("s") * 2 + lax.axis_index("c")
        qq = wid // 2
        b0 = (wid % 2) * lanes
        idx_copy = pltpu.async_copy(idx_hbm.at[qq, pl.ds(b0, lanes)], idx_v, isem)

        lane_iota = lax.iota(jnp.int32, lanes)
        idx_copy.wait()
        iv = jnp.maximum(idx_v[...], 0)
        for j in range(lanes):
            m = iv[j]
            chunk_v[j, pl.ds(0, lanes)] = (
                1 - jnp.minimum(jnp.abs(lane_iota - m % lanes), 1)
            ).astype(jnp.float32)

        copies = [
            pltpu.async_copy(
                chunk_v.at[j],
                out_hbm.at[b0 + j, qq, pl.ds((iv[j] // lanes) * lanes, lanes)],
                csem,
            )
            for j in range(lanes)
        ]
        for cp in copies:
            cp.wait()

    return sc_kernel


def kernel(keys, queries, step, provided_attention):
    batch_size, n_queries, _ = queries.shape
    n_keys = keys.shape[1]
    lanes = 16

    idx_t = provided_attention.T.astype(jnp.int32)
    acc = jax.new_ref(jnp.zeros((batch_size, n_queries, n_keys), jnp.float32))
    _build_sc_scatter(batch_size, n_queries, n_keys, lanes)(idx_t, acc)
    return acc[...]
